# baseline (speedup 1.0000x reference)
import math

import jax
import jax.numpy as jnp
import numpy as np
from jax.experimental import pallas as pl
from jax.experimental.pallas import tpu as pltpu

_ORDER = 13
_SEED = 48879


def _hilbert_encode_vec(x, y, order):
    x = x.astype(np.int64).copy()
    y = y.astype(np.int64).copy()
    d = np.zeros_like(x)
    s = 1 << (order - 1)
    while s > 0:
        rx = ((x & s) > 0).astype(np.int64)
        ry = ((y & s) > 0).astype(np.int64)
        d += s * s * ((3 * rx) ^ ry)
        swap = ry == 0
        flip = swap & (rx == 1)
        x_f = np.where(flip, s - 1 - x, x)
        y_f = np.where(flip, s - 1 - y, y)
        x_new = np.where(swap, y_f, x_f)
        y_new = np.where(swap, x_f, y_f)
        x, y = x_new, y_new
        s >>= 1
    return d


def _v_mask_generative(addr_u64, rounds, seed):
    h = addr_u64 ^ np.uint64(seed & 0xFFFFFFFFFFFFFFFF)
    for _ in range(rounds):
        h = h * np.uint64(6364136223846793005) + np.uint64(1442695040888963407)
        h = h ^ (h >> np.uint64(33))
    frac = (h & np.uint64(0xFFFFFF)).astype(np.float64) / float(0xFFFFFF)
    return (frac * 2.0 - 1.0).astype(np.float32)


def _resonance_vec(S, D, seed_val, order):
    i = np.arange(S, dtype=np.int64)
    j = i.copy()
    t = _hilbert_encode_vec(i, j, order)
    addr = (i.astype(np.uint64) << np.uint64(32)) | j.astype(np.uint64)
    s_long = int(round(seed_val))
    ground_weight = _v_mask_generative(addr, 4, s_long ^ D)
    sig = (np.uint64(s_long) ^ np.uint64(D) ^ t.astype(np.uint64)) & np.uint64(0xFFFFFFFF)
    phase = (sig % np.uint64(1000)).astype(np.float64) / 1000.0 * 2.0 * math.pi
    resonance = np.sin(phase).astype(np.float32)
    return ground_weight * resonance


def _scale_body(scale_ref, res_ref, q_ref, o_ref):
    s = scale_ref[0, 0]
    o_ref[:, :] = q_ref[:, :] * (res_ref[:, :] * s)


def kernel(q, k, v_val, latent_seed):
    B, S, D = q.shape
    res = _resonance_vec(S, D, float(_SEED), _ORDER)
    res_full = jnp.asarray(np.tile(res, B).reshape(B * S, 1))
    scale = (latent_seed * jnp.float32(1.0 / _SEED)).reshape(1, 1)

    rows = B * S
    block_rows = 512
    grid = rows // block_rows

    q2 = q.reshape(rows, D)
    out = pl.pallas_call(
        _scale_body,
        grid=(grid,),
        in_specs=[
            pl.BlockSpec(memory_space=pltpu.SMEM),
            pl.BlockSpec((block_rows, 1), lambda i: (i, 0)),
            pl.BlockSpec((block_rows, D), lambda i: (i, 0)),
        ],
        out_specs=pl.BlockSpec((block_rows, D), lambda i: (i, 0)),
        out_shape=jax.ShapeDtypeStruct((rows, D), jnp.float32),
    )(scale, res_full, q2)
    return out.reshape(B, S, D)


# speedup vs baseline: 1.0947x; 1.0947x over previous
import math

import jax
import jax.numpy as jnp
import numpy as np
from jax.experimental import pallas as pl
from jax.experimental.pallas import tpu as pltpu

_ORDER = 13
_SEED = 48879


def _hilbert_encode_vec(x, y, order):
    x = x.astype(np.int64).copy()
    y = y.astype(np.int64).copy()
    d = np.zeros_like(x)
    s = 1 << (order - 1)
    while s > 0:
        rx = ((x & s) > 0).astype(np.int64)
        ry = ((y & s) > 0).astype(np.int64)
        d += s * s * ((3 * rx) ^ ry)
        swap = ry == 0
        flip = swap & (rx == 1)
        x_f = np.where(flip, s - 1 - x, x)
        y_f = np.where(flip, s - 1 - y, y)
        x_new = np.where(swap, y_f, x_f)
        y_new = np.where(swap, x_f, y_f)
        x, y = x_new, y_new
        s >>= 1
    return d


def _v_mask_generative(addr_u64, rounds, seed):
    h = addr_u64 ^ np.uint64(seed & 0xFFFFFFFFFFFFFFFF)
    for _ in range(rounds):
        h = h * np.uint64(6364136223846793005) + np.uint64(1442695040888963407)
        h = h ^ (h >> np.uint64(33))
    frac = (h & np.uint64(0xFFFFFF)).astype(np.float64) / float(0xFFFFFF)
    return (frac * 2.0 - 1.0).astype(np.float32)


def _resonance_vec(S, D, seed_val, order):
    i = np.arange(S, dtype=np.int64)
    j = i.copy()
    t = _hilbert_encode_vec(i, j, order)
    addr = (i.astype(np.uint64) << np.uint64(32)) | j.astype(np.uint64)
    s_long = int(round(seed_val))
    ground_weight = _v_mask_generative(addr, 4, s_long ^ D)
    sig = (np.uint64(s_long) ^ np.uint64(D) ^ t.astype(np.uint64)) & np.uint64(0xFFFFFFFF)
    phase = (sig % np.uint64(1000)).astype(np.float64) / 1000.0 * 2.0 * math.pi
    resonance = np.sin(phase).astype(np.float32)
    return ground_weight * resonance


def _scale_body(scale_ref, res_ref, q_ref, o_ref):
    s = scale_ref[0, 0]
    o_ref[:, :] = q_ref[:, :] * (res_ref[:, :] * s)


def kernel(q, k, v_val, latent_seed):
    B, S, D = q.shape
    res = _resonance_vec(S, D, float(_SEED), _ORDER)
    res_full = jnp.asarray(np.tile(res, B).reshape(B * S, 1))
    scale = (latent_seed * jnp.float32(1.0 / _SEED)).reshape(1, 1)

    rows = B * S
    block_rows = 2048
    grid = rows // block_rows

    q2 = q.reshape(rows, D)
    out = pl.pallas_call(
        _scale_body,
        grid=(grid,),
        in_specs=[
            pl.BlockSpec(memory_space=pltpu.SMEM),
            pl.BlockSpec((block_rows, 1), lambda i: (i, 0)),
            pl.BlockSpec((block_rows, D), lambda i: (i, 0)),
        ],
        out_specs=pl.BlockSpec((block_rows, D), lambda i: (i, 0)),
        out_shape=jax.ShapeDtypeStruct((rows, D), jnp.float32),
    )(scale, res_full, q2)
    return out.reshape(B, S, D)


# speedup vs baseline: 1.1177x; 1.0210x over previous
import math

import jax
import jax.numpy as jnp
import numpy as np
from jax.experimental import pallas as pl
from jax.experimental.pallas import tpu as pltpu

_ORDER = 13
_SEED = 48879


def _hilbert_encode_vec(x, y, order):
    x = x.astype(np.int64).copy()
    y = y.astype(np.int64).copy()
    d = np.zeros_like(x)
    s = 1 << (order - 1)
    while s > 0:
        rx = ((x & s) > 0).astype(np.int64)
        ry = ((y & s) > 0).astype(np.int64)
        d += s * s * ((3 * rx) ^ ry)
        swap = ry == 0
        flip = swap & (rx == 1)
        x_f = np.where(flip, s - 1 - x, x)
        y_f = np.where(flip, s - 1 - y, y)
        x_new = np.where(swap, y_f, x_f)
        y_new = np.where(swap, x_f, y_f)
        x, y = x_new, y_new
        s >>= 1
    return d


def _v_mask_generative(addr_u64, rounds, seed):
    h = addr_u64 ^ np.uint64(seed & 0xFFFFFFFFFFFFFFFF)
    for _ in range(rounds):
        h = h * np.uint64(6364136223846793005) + np.uint64(1442695040888963407)
        h = h ^ (h >> np.uint64(33))
    frac = (h & np.uint64(0xFFFFFF)).astype(np.float64) / float(0xFFFFFF)
    return (frac * 2.0 - 1.0).astype(np.float32)


def _resonance_vec(S, D, seed_val, order):
    i = np.arange(S, dtype=np.int64)
    j = i.copy()
    t = _hilbert_encode_vec(i, j, order)
    addr = (i.astype(np.uint64) << np.uint64(32)) | j.astype(np.uint64)
    s_long = int(round(seed_val))
    ground_weight = _v_mask_generative(addr, 4, s_long ^ D)
    sig = (np.uint64(s_long) ^ np.uint64(D) ^ t.astype(np.uint64)) & np.uint64(0xFFFFFFFF)
    phase = (sig % np.uint64(1000)).astype(np.float64) / 1000.0 * 2.0 * math.pi
    resonance = np.sin(phase).astype(np.float32)
    return ground_weight * resonance


def _scale_body(block_rows, scale_ref, res_ref, q_ref, o_ref):
    i = pl.program_id(0)
    s = scale_ref[0, 0]
    r = res_ref[pl.ds(i * block_rows, block_rows), :]
    o_ref[:, :] = q_ref[:, :] * (r * s)


def kernel(q, k, v_val, latent_seed):
    B, S, D = q.shape
    res = _resonance_vec(S, D, float(_SEED), _ORDER)
    res_full = jnp.asarray(np.tile(res, B).reshape(B * S, 1))
    scale = (latent_seed * jnp.float32(1.0 / _SEED)).reshape(1, 1)

    rows = B * S
    block_rows = 2048
    grid = rows // block_rows

    q2 = q.reshape(rows, D)
    import functools
    out = pl.pallas_call(
        functools.partial(_scale_body, block_rows),
        grid=(grid,),
        in_specs=[
            pl.BlockSpec(memory_space=pltpu.SMEM),
            pl.BlockSpec((rows, 1), lambda i: (0, 0)),
            pl.BlockSpec((block_rows, D), lambda i: (i, 0)),
        ],
        out_specs=pl.BlockSpec((block_rows, D), lambda i: (i, 0)),
        out_shape=jax.ShapeDtypeStruct((rows, D), jnp.float32),
    )(scale, res_full, q2)
    return out.reshape(B, S, D)
